# baseline (speedup 1.0000x reference)
import functools

import numpy as np
import jax
import jax.numpy as jnp
from jax import lax
from jax.experimental import pallas as pl
from jax.experimental.pallas import tpu as pltpu
from jax.experimental.pallas import tpu_sc as plsc

_Z = 128
_D = 2 * _Z
_NW = 32
_CHUNK = 256
_ROWS = 512


_ROTS = ((13, 15, 26, 6), (17, 29, 16, 24))


def _np_threefry2x32(k0, k1, x0, x1):
    k0 = np.uint32(k0); k1 = np.uint32(k1)
    ks = [k0, k1, np.uint32(k0 ^ k1 ^ np.uint32(0x1BD11BDA))]
    x0 = np.asarray(x0, np.uint32) + ks[0]
    x1 = np.asarray(x1, np.uint32) + ks[1]
    with np.errstate(over="ignore"):
        for i in range(5):
            for r in _ROTS[i % 2]:
                x0 = x0 + x1
                x1 = (x1 << np.uint32(r)) | (x1 >> np.uint32(32 - r))
                x1 = x1 ^ x0
            x0 = x0 + ks[(i + 1) % 3]
            x1 = x1 + ks[(i + 2) % 3] + np.uint32(i + 1)
    return x0, x1


def _np_subkeys():
    b1, b2 = _np_threefry2x32(0, 42, np.zeros(3, np.uint32),
                              np.arange(3, dtype=np.uint32))
    return [(int(b1[j]), int(b2[j])) for j in range(3)]


_SUBKEYS = _np_subkeys()


def _tf_bits(k0_int, k1_int, idx_u32):
    k0 = jnp.uint32(k0_int)
    k1 = jnp.uint32(k1_int)
    ks = (k0, k1, jnp.uint32(k0_int ^ k1_int ^ 0x1BD11BDA))
    x0 = jnp.full(idx_u32.shape, ks[0], jnp.uint32)
    x1 = idx_u32 + ks[1]
    for i in range(5):
        for r in _ROTS[i % 2]:
            x0 = x0 + x1
            x1 = (x1 << jnp.uint32(r)) | (x1 >> jnp.uint32(32 - r))
            x1 = x1 ^ x0
        x0 = x0 + ks[(i + 1) % 3]
        x1 = x1 + ks[(i + 2) % 3] + jnp.uint32(i + 1)
    return x0 ^ x1


_U_LO = np.float32(np.nextafter(np.float32(-1.0), np.float32(0.0)))

_CC = (1.2533326, 0.32776460, 0.017342027, -0.0042949238,
       0.00022931020, 2.9562478e-06)
_CT = (2.1027328, -1.9204562, 1.9148134, -0.55467651,
       0.081071377, -0.0047685542)


def _bits_to_normal(bits):
    fb = (bits >> jnp.uint32(9)) | jnp.uint32(0x3F800000)
    f = lax.bitcast_convert_type(fb, jnp.float32) - jnp.float32(1.0)
    u = jnp.maximum(jnp.float32(_U_LO),
                    f * (jnp.float32(1.0) - _U_LO) + _U_LO)
    w = -jnp.log(jnp.float32(1.0) - u * u)
    cen = w < jnp.float32(5.0)
    t = jnp.where(cen, w, jnp.sqrt(w))
    p = jnp.where(cen, jnp.float32(_CC[5]), jnp.float32(_CT[5]))
    for i in range(4, -1, -1):
        p = p * t + jnp.where(cen, jnp.float32(_CC[i]), jnp.float32(_CT[i]))
    return u * p


def _sc_gather(s_idx, p_idx, o_idx, e_table, r_table, chunk_rows):
    n = s_idx.shape[0]
    per_w = n // _NW
    cpt = per_w // chunk_rows
    mesh = plsc.VectorSubcoreMesh(core_axis_name="c", subcore_axis_name="s")

    @functools.partial(
        pl.kernel,
        mesh=mesh,
        out_type=[jax.ShapeDtypeStruct((n, _D), jnp.float32)] * 3,
        scratch_types=[
            pltpu.VMEM((per_w,), jnp.int32),
            pltpu.VMEM((per_w,), jnp.int32),
            pltpu.VMEM((per_w,), jnp.int32),
            pltpu.VMEM((chunk_rows, _D), jnp.float32),
            pltpu.VMEM((chunk_rows, _D), jnp.float32),
            pltpu.SemaphoreType.DMA,
            pltpu.SemaphoreType.DMA,
            pltpu.SemaphoreType.DMA,
            pltpu.SemaphoreType.DMA,
        ],
    )
    def gather_kernel(s_hbm, p_hbm, o_hbm, et_hbm, rt_hbm,
                      gs_hbm, gp_hbm, go_hbm, idx_s, idx_p, idx_o,
                      rows0, rows1, sg0, sg1, so0, so1):
        wid = lax.axis_index("s") * 2 + lax.axis_index("c")
        base = wid * per_w
        idx_v = (idx_s, idx_p, idx_o)
        for t, ih in enumerate((s_hbm, p_hbm, o_hbm)):
            pltpu.sync_copy(ih.at[pl.ds(base, per_w)], idx_v[t])

        tables = (et_hbm, rt_hbm, et_hbm)
        outs = (gs_hbm, gp_hbm, go_hbm)
        order = [(t, c) for t in range(3) for c in range(cpt)]
        rows = (rows0, rows1)
        sg = (sg0, sg1)
        so = (so0, so1)
        num = len(order)

        def start_gather(k, b):
            t, c = order[k]
            idx = idx_v[t].at[pl.ds(c * chunk_rows, chunk_rows)]
            return pltpu.async_copy(tables[t].at[idx], rows[b], sg[b])

        def start_store(k, b):
            t, c = order[k]
            off = base + c * chunk_rows
            return pltpu.async_copy(rows[b], outs[t].at[pl.ds(off, chunk_rows)],
                                    so[b])

        hg = [None] * num
        hs = [None] * num
        hg[0] = start_gather(0, 0)
        for k in range(num):
            b = k & 1
            hg[k].wait()
            hs[k] = start_store(k, b)
            if k + 1 < num:
                if k >= 1:
                    hs[k - 1].wait()
                hg[k + 1] = start_gather(k + 1, 1 - b)
        if num >= 2:
            hs[num - 2].wait()
        hs[num - 1].wait()

    return gather_kernel(s_idx, p_idx, o_idx, e_table, r_table)


def _score_body(i0_ref, gs_ref, gp_ref, go_ref, out_ref, *, base_rows):
    g = pl.program_id(0)
    base = (g * (_ROWS * _Z) + base_rows * _Z).astype(jnp.uint32)
    i = base + i0_ref[...]

    def z_sample(ref, kpair):
        eps = _bits_to_normal(_tf_bits(kpair[0], kpair[1], i))
        mean = ref[:, :_Z]
        logvar = ref[:, _Z:]
        return eps * jnp.exp(logvar * jnp.float32(0.5)) + mean

    zs = z_sample(gs_ref, _SUBKEYS[0])
    zp = z_sample(gp_ref, _SUBKEYS[1])
    zo = z_sample(go_ref, _SUBKEYS[2])
    ones = jnp.ones((_Z, 1), jnp.float32)
    out_ref[...] = lax.dot_general(zs * zp * zo, ones,
                                   (((1,), (0,)), ((), ())),
                                   precision=lax.Precision.HIGHEST,
                                   preferred_element_type=jnp.float32)


def _tc_score_offset(i0, gs, gp, go, base_rows):
    n = gs.shape[0]
    grid = n // _ROWS
    return pl.pallas_call(
        functools.partial(_score_body, base_rows=base_rows),
        grid=(grid,),
        in_specs=[pl.BlockSpec((_ROWS, _Z), lambda g: (0, 0))]
        + [pl.BlockSpec((_ROWS, _D), lambda g: (g, 0))] * 3,
        out_specs=pl.BlockSpec((_ROWS, 1), lambda g: (g, 0)),
        out_shape=jax.ShapeDtypeStruct((n, 1), jnp.float32),
    )(i0, gs, gp, go)


_NSPLIT = 4


def kernel(s, p, o, e_table, r_table):
    B, L = s.shape
    n = B * L
    s_flat = s.reshape(n).astype(jnp.int32)
    p_flat = p.reshape(n).astype(jnp.int32)
    o_flat = o.reshape(n).astype(jnp.int32)
    step = n // _NSPLIT
    chunk_rows = step // _NW // 8
    i0 = (jnp.arange(_ROWS * _Z, dtype=jnp.uint32)
          .reshape(_ROWS, _Z))
    gathered = []
    for k in range(_NSPLIT):
        sl = slice(k * step, (k + 1) * step)
        gathered.append(_sc_gather(s_flat[sl], p_flat[sl], o_flat[sl],
                                   e_table, r_table, chunk_rows))
    outs = [_tc_score_offset(i0, gs, gp, go, k * step)
            for k, (gs, gp, go) in enumerate(gathered)]
    scores = jnp.concatenate(outs, axis=0)
    return scores.reshape(B, L)


# speedup vs baseline: 1.0195x; 1.0195x over previous
import functools

import numpy as np
import jax
import jax.numpy as jnp
from jax import lax
from jax.experimental import pallas as pl
from jax.experimental.pallas import tpu as pltpu
from jax.experimental.pallas import tpu_sc as plsc

_Z = 128
_D = 2 * _Z
_NW = 32
_CHUNK = 256
_ROWS = 512


_ROTS = ((13, 15, 26, 6), (17, 29, 16, 24))


def _np_threefry2x32(k0, k1, x0, x1):
    k0 = np.uint32(k0); k1 = np.uint32(k1)
    ks = [k0, k1, np.uint32(k0 ^ k1 ^ np.uint32(0x1BD11BDA))]
    x0 = np.asarray(x0, np.uint32) + ks[0]
    x1 = np.asarray(x1, np.uint32) + ks[1]
    with np.errstate(over="ignore"):
        for i in range(5):
            for r in _ROTS[i % 2]:
                x0 = x0 + x1
                x1 = (x1 << np.uint32(r)) | (x1 >> np.uint32(32 - r))
                x1 = x1 ^ x0
            x0 = x0 + ks[(i + 1) % 3]
            x1 = x1 + ks[(i + 2) % 3] + np.uint32(i + 1)
    return x0, x1


def _np_subkeys():
    b1, b2 = _np_threefry2x32(0, 42, np.zeros(3, np.uint32),
                              np.arange(3, dtype=np.uint32))
    return [(int(b1[j]), int(b2[j])) for j in range(3)]


_SUBKEYS = _np_subkeys()


def _tf_bits(k0_int, k1_int, idx_u32):
    k0 = jnp.uint32(k0_int)
    k1 = jnp.uint32(k1_int)
    ks = (k0, k1, jnp.uint32(k0_int ^ k1_int ^ 0x1BD11BDA))
    x0 = jnp.full(idx_u32.shape, ks[0], jnp.uint32)
    x1 = idx_u32 + ks[1]
    for i in range(5):
        for r in _ROTS[i % 2]:
            x0 = x0 + x1
            x1 = (x1 << jnp.uint32(r)) | (x1 >> jnp.uint32(32 - r))
            x1 = x1 ^ x0
        x0 = x0 + ks[(i + 1) % 3]
        x1 = x1 + ks[(i + 2) % 3] + jnp.uint32(i + 1)
    return x0 ^ x1


_U_LO = np.float32(np.nextafter(np.float32(-1.0), np.float32(0.0)))

_CC = (1.2533326, 0.32776460, 0.017342027, -0.0042949238,
       0.00022931020, 2.9562478e-06)
_CT = (2.1027328, -1.9204562, 1.9148134, -0.55467651,
       0.081071377, -0.0047685542)


def _bits_to_normal(bits):
    fb = (bits >> jnp.uint32(9)) | jnp.uint32(0x3F800000)
    f = lax.bitcast_convert_type(fb, jnp.float32) - jnp.float32(1.0)
    u = jnp.maximum(jnp.float32(_U_LO),
                    f * (jnp.float32(1.0) - _U_LO) + _U_LO)
    w = -jnp.log(jnp.float32(1.0) - u * u)
    cen = w < jnp.float32(5.0)
    t = jnp.where(cen, w, jnp.sqrt(w))
    p = jnp.where(cen, jnp.float32(_CC[5]), jnp.float32(_CT[5]))
    for i in range(4, -1, -1):
        p = p * t + jnp.where(cen, jnp.float32(_CC[i]), jnp.float32(_CT[i]))
    return u * p


def _sc_gather(s_idx, p_idx, o_idx, e_table, r_table, chunk_rows):
    n = s_idx.shape[0]
    per_w = n // _NW
    cpt = per_w // chunk_rows
    mesh = plsc.VectorSubcoreMesh(core_axis_name="c", subcore_axis_name="s")

    @functools.partial(
        pl.kernel,
        mesh=mesh,
        out_type=[jax.ShapeDtypeStruct((n, _D), jnp.float32)] * 3,
        scratch_types=[
            pltpu.VMEM((per_w,), jnp.int32),
            pltpu.VMEM((per_w,), jnp.int32),
            pltpu.VMEM((per_w,), jnp.int32),
            pltpu.VMEM((chunk_rows, _D), jnp.float32),
            pltpu.VMEM((chunk_rows, _D), jnp.float32),
            pltpu.SemaphoreType.DMA,
            pltpu.SemaphoreType.DMA,
            pltpu.SemaphoreType.DMA,
            pltpu.SemaphoreType.DMA,
        ],
    )
    def gather_kernel(s_hbm, p_hbm, o_hbm, et_hbm, rt_hbm,
                      gs_hbm, gp_hbm, go_hbm, idx_s, idx_p, idx_o,
                      rows0, rows1, sg0, sg1, so0, so1):
        wid = lax.axis_index("s") * 2 + lax.axis_index("c")
        base = wid * per_w
        idx_v = (idx_s, idx_p, idx_o)
        for t, ih in enumerate((s_hbm, p_hbm, o_hbm)):
            pltpu.sync_copy(ih.at[pl.ds(base, per_w)], idx_v[t])

        tables = (et_hbm, rt_hbm, et_hbm)
        outs = (gs_hbm, gp_hbm, go_hbm)
        order = [(t, c) for t in range(3) for c in range(cpt)]
        rows = (rows0, rows1)
        sg = (sg0, sg1)
        so = (so0, so1)
        num = len(order)

        def start_gather(k, b):
            t, c = order[k]
            idx = idx_v[t].at[pl.ds(c * chunk_rows, chunk_rows)]
            return pltpu.async_copy(tables[t].at[idx], rows[b], sg[b])

        def start_store(k, b):
            t, c = order[k]
            off = base + c * chunk_rows
            return pltpu.async_copy(rows[b], outs[t].at[pl.ds(off, chunk_rows)],
                                    so[b])

        hg = [None] * num
        hs = [None] * num
        hg[0] = start_gather(0, 0)
        for k in range(num):
            b = k & 1
            hg[k].wait()
            hs[k] = start_store(k, b)
            if k + 1 < num:
                if k >= 1:
                    hs[k - 1].wait()
                hg[k + 1] = start_gather(k + 1, 1 - b)
        if num >= 2:
            hs[num - 2].wait()
        hs[num - 1].wait()

    return gather_kernel(s_idx, p_idx, o_idx, e_table, r_table)


def _score_body(i0_ref, gs_ref, gp_ref, go_ref, out_ref, *, base_rows):
    g = pl.program_id(0)
    base = (g * (_ROWS * _Z) + base_rows * _Z).astype(jnp.uint32)
    i = base + i0_ref[...]

    def z_sample(ref, kpair):
        eps = _bits_to_normal(_tf_bits(kpair[0], kpair[1], i))
        mean = ref[:, :_Z]
        logvar = ref[:, _Z:]
        return eps * jnp.exp(logvar * jnp.float32(0.5)) + mean

    zs = z_sample(gs_ref, _SUBKEYS[0])
    zp = z_sample(gp_ref, _SUBKEYS[1])
    zo = z_sample(go_ref, _SUBKEYS[2])
    ones = jnp.ones((_Z, 1), jnp.float32)
    out_ref[...] = lax.dot_general(zs * zp * zo, ones,
                                   (((1,), (0,)), ((), ())),
                                   precision=lax.Precision.HIGHEST,
                                   preferred_element_type=jnp.float32)


def _tc_score_offset(i0, gs, gp, go, base_rows):
    n = gs.shape[0]
    grid = n // _ROWS
    return pl.pallas_call(
        functools.partial(_score_body, base_rows=base_rows),
        grid=(grid,),
        in_specs=[pl.BlockSpec((_ROWS, _Z), lambda g: (0, 0))]
        + [pl.BlockSpec((_ROWS, _D), lambda g: (g, 0))] * 3,
        out_specs=pl.BlockSpec((_ROWS, 1), lambda g: (g, 0)),
        out_shape=jax.ShapeDtypeStruct((n, 1), jnp.float32),
    )(i0, gs, gp, go)


_NSPLIT = 8


def kernel(s, p, o, e_table, r_table):
    B, L = s.shape
    n = B * L
    s_flat = s.reshape(n).astype(jnp.int32)
    p_flat = p.reshape(n).astype(jnp.int32)
    o_flat = o.reshape(n).astype(jnp.int32)
    step = n // _NSPLIT
    chunk_rows = step // _NW // 4
    i0 = (jnp.arange(_ROWS * _Z, dtype=jnp.uint32)
          .reshape(_ROWS, _Z))
    gathered = []
    for k in range(_NSPLIT):
        sl = slice(k * step, (k + 1) * step)
        gathered.append(_sc_gather(s_flat[sl], p_flat[sl], o_flat[sl],
                                   e_table, r_table, chunk_rows))
    outs = [_tc_score_offset(i0, gs, gp, go, k * step)
            for k, (gs, gp, go) in enumerate(gathered)]
    scores = jnp.concatenate(outs, axis=0)
    return scores.reshape(B, L)
